# Initial kernel scaffold; baseline (speedup 1.0000x reference)
import jax
import jax.numpy as jnp
from jax.experimental import pallas as pl


def kernel(x, edge_index, W1, b1, W2, b2, Wc, bc):
    raise NotImplementedError("write your pallas kernel here")


# speedup vs baseline: 8.1156x; 8.1156x over previous
import functools

import jax
import jax.numpy as jnp
from jax import lax
from jax.experimental import pallas as pl
from jax.experimental.pallas import tpu as pltpu
from jax.experimental.pallas import tpu_sc as plsc

NN = 50000
NP = 50048
EE = 1600000
EP = 1605632
RT = 3128
CH = 128
NJF = 784
NJH = 392
FH1 = 256
FH2 = 128
NCLS = 10


@functools.lru_cache(maxsize=None)
def _deg_kernel():
    mesh = plsc.VectorSubcoreMesh(core_axis_name="c", subcore_axis_name="s")

    SB = 56

    @functools.partial(
        pl.kernel,
        out_type=jax.ShapeDtypeStruct((2, NP, 16), jnp.float32),
        mesh=mesh,
        compiler_params=pltpu.CompilerParams(use_tc_tiling_on_sc=False),
        scratch_types=[
            pltpu.VMEM((SB, CH), jnp.int32),
            pltpu.VMEM((CH, 16), jnp.float32),
            pltpu.VMEM_SHARED((NP, 16), jnp.float32),
        ],
    )
    def deg(src_hbm, dst_hbm, zrow_hbm, ones_hbm, out_hbm, idx_v, ones_v, acc_sh):
        c = lax.axis_index("c")
        s = lax.axis_index("s")
        row0 = s * RT
        pltpu.sync_copy(zrow_hbm, acc_sh.at[pl.ds(row0, RT)])
        pltpu.sync_copy(ones_hbm, ones_v)
        plsc.subcore_barrier()

        def blk(b, carry):
            @pl.when(c == 0)
            def _():
                pltpu.sync_copy(src_hbm.at[s, pl.ds(b * SB, SB)], idx_v)

            @pl.when(c == 1)
            def _():
                pltpu.sync_copy(dst_hbm.at[s, pl.ds(b * SB, SB)], idx_v)

            def body(j, carry2):
                pltpu.sync_copy(ones_v, acc_sh.at[idx_v.at[j]], add=True)
                return carry2

            return lax.fori_loop(0, SB, body, carry)

        lax.fori_loop(0, NJF // SB, blk, 0)
        plsc.subcore_barrier()
        pltpu.sync_copy(acc_sh.at[pl.ds(row0, RT)], out_hbm.at[c, pl.ds(row0, RT)])

    return deg


@functools.lru_cache(maxsize=None)
def _agg1_kernel():
    mesh = plsc.VectorSubcoreMesh(core_axis_name="c", subcore_axis_name="s")

    SB = 56

    @functools.partial(
        pl.kernel,
        out_type=jax.ShapeDtypeStruct((2, NP, 16), jnp.float32),
        mesh=mesh,
        compiler_params=pltpu.CompilerParams(use_tc_tiling_on_sc=False),
        scratch_types=[
            pltpu.VMEM((SB, CH), jnp.int32),
            pltpu.VMEM((SB, CH), jnp.int32),
            pltpu.VMEM((CH, 16), jnp.float32),
            pltpu.VMEM_SHARED((NP, 16), jnp.float32),
        ],
    )
    def agg1(xn_hbm, src_hbm, dst_hbm, zrow_hbm, out_hbm,
             src_v, dst_v, row_v, acc_sh):
        c = lax.axis_index("c")
        s = lax.axis_index("s")
        row0 = s * RT
        pltpu.sync_copy(zrow_hbm, acc_sh.at[pl.ds(row0, RT)])
        plsc.subcore_barrier()

        def blk(b, carry):
            pltpu.sync_copy(src_hbm.at[c, s, pl.ds(b * SB, SB)], src_v)
            pltpu.sync_copy(dst_hbm.at[c, s, pl.ds(b * SB, SB)], dst_v)

            def body(j, carry2):
                pltpu.sync_copy(xn_hbm.at[src_v.at[j]], row_v)
                pltpu.sync_copy(row_v, acc_sh.at[dst_v.at[j]], add=True)
                return carry2

            return lax.fori_loop(0, SB, body, carry)

        lax.fori_loop(0, NJH // SB, blk, 0)
        plsc.subcore_barrier()
        pltpu.sync_copy(acc_sh.at[pl.ds(row0, RT)], out_hbm.at[c, pl.ds(row0, RT)])

    return agg1


@functools.lru_cache(maxsize=None)
def _agg2_kernel():
    mesh = plsc.VectorSubcoreMesh(core_axis_name="c", subcore_axis_name="s")
    oshape = tuple(jax.ShapeDtypeStruct((NP, 32), jnp.float32) for _ in range(4))

    @functools.partial(
        pl.kernel,
        out_type=oshape,
        mesh=mesh,
        compiler_params=pltpu.CompilerParams(use_tc_tiling_on_sc=False),
        scratch_types=[
            pltpu.VMEM((16, CH), jnp.int32),
            pltpu.VMEM((16, CH), jnp.int32),
            pltpu.VMEM((CH, 32), jnp.float32),
            pltpu.VMEM_SHARED((NP, 32), jnp.float32),
        ],
    )
    def agg2(t0, t1, t2, t3, src_hbm, dst_hbm, zrow_hbm,
             o0, o1, o2, o3, src_v, dst_v, row_v, acc_sh):
        SB = 16
        c = lax.axis_index("c")
        s = lax.axis_index("s")
        row0 = s * RT
        t_refs = (t0, t1, t2, t3)
        o_refs = (o0, o1, o2, o3)
        for cc in range(2):
            @pl.when(c == cc)
            def _(cc=cc):
                for kk in range(2):
                    k = 2 * cc + kk
                    pltpu.sync_copy(zrow_hbm, acc_sh.at[pl.ds(row0, RT)])
                    plsc.subcore_barrier()

                    def blk(b, carry, k=k):
                        pltpu.sync_copy(src_hbm.at[s, pl.ds(b * SB, SB)], src_v)
                        pltpu.sync_copy(dst_hbm.at[s, pl.ds(b * SB, SB)], dst_v)

                        def body(j, carry2):
                            pltpu.sync_copy(t_refs[k].at[src_v.at[j]], row_v)
                            pltpu.sync_copy(row_v, acc_sh.at[dst_v.at[j]], add=True)
                            return carry2

                        return lax.fori_loop(0, SB, body, carry)

                    lax.fori_loop(0, NJF // SB, blk, 0)
                    plsc.subcore_barrier()
                    pltpu.sync_copy(acc_sh.at[pl.ds(row0, RT)],
                                    o_refs[k].at[pl.ds(row0, RT)])
                    plsc.subcore_barrier()

    return agg2


def _prep_body(degs_ref, x_ref, xn_ref, ns_ref, nd_ref):
    ns = lax.rsqrt(jnp.maximum(degs_ref[0][:, 0:1], 1.0))
    nd = lax.rsqrt(jnp.maximum(degs_ref[1][:, 0:1], 1.0))
    xn_ref[...] = x_ref[...] * ns
    ns_ref[...] = ns
    nd_ref[...] = nd


def _prep_call(degs, xpad):
    return pl.pallas_call(
        _prep_body,
        grid=(16,),
        in_specs=[
            pl.BlockSpec((2, RT, 16), lambda i: (0, i, 0)),
            pl.BlockSpec((RT, 16), lambda i: (i, 0)),
        ],
        out_specs=[
            pl.BlockSpec((RT, 16), lambda i: (i, 0)),
            pl.BlockSpec((RT, 1), lambda i: (i, 0)),
            pl.BlockSpec((RT, 1), lambda i: (i, 0)),
        ],
        out_shape=[
            jax.ShapeDtypeStruct((NP, 16), jnp.float32),
            jax.ShapeDtypeStruct((NP, 1), jnp.float32),
            jax.ShapeDtypeStruct((NP, 1), jnp.float32),
        ],
    )(degs, xpad)


def _dense_body(aggp_ref, ns_ref, nd_ref, w1_ref, b1_ref, w2_ref,
                t0_ref, t1_ref, t2_ref, t3_ref):
    agg = (aggp_ref[0] + aggp_ref[1]) * nd_ref[...]
    h1 = jnp.dot(agg, w1_ref[...], preferred_element_type=jnp.float32,
                 precision=lax.Precision.HIGHEST)
    h1 = jnp.maximum(h1 + b1_ref[...], 0.0)
    h1n = h1 * ns_ref[...]
    t = jnp.dot(h1n, w2_ref[...], preferred_element_type=jnp.float32,
                precision=lax.Precision.HIGHEST)
    t0_ref[...] = t[:, 0:32]
    t1_ref[...] = t[:, 32:64]
    t2_ref[...] = t[:, 64:96]
    t3_ref[...] = t[:, 96:128]


def _dense_call(aggp, ns, nd, w1p, b1r, W2):
    return pl.pallas_call(
        _dense_body,
        grid=(16,),
        in_specs=[
            pl.BlockSpec((2, RT, 16), lambda i: (0, i, 0)),
            pl.BlockSpec((RT, 1), lambda i: (i, 0)),
            pl.BlockSpec((RT, 1), lambda i: (i, 0)),
            pl.BlockSpec((16, FH1), lambda i: (0, 0)),
            pl.BlockSpec((1, FH1), lambda i: (0, 0)),
            pl.BlockSpec((FH1, FH2), lambda i: (0, 0)),
        ],
        out_specs=[pl.BlockSpec((RT, 32), lambda i: (i, 0)) for _ in range(4)],
        out_shape=[jax.ShapeDtypeStruct((NP, 32), jnp.float32) for _ in range(4)],
    )(aggp, ns, nd, w1p, b1r, W2)


def _final_body(a0, a1, a2, a3, nd_ref, b2_ref, wc_ref, bc_ref, out_ref, acc):
    i = pl.program_id(0)
    h = jnp.concatenate([a0[...], a1[...], a2[...], a3[...]], axis=1)
    h2 = jnp.maximum(h * nd_ref[...] + b2_ref[...], 0.0)
    rows = RT * i + lax.broadcasted_iota(jnp.int32, (RT, 1), 0)
    h2 = jnp.where(rows < NN, h2, 0.0)
    part = jnp.sum(h2, axis=0, keepdims=True)

    @pl.when(i == 0)
    def _():
        acc[...] = part

    @pl.when(i > 0)
    def _():
        acc[...] = acc[...] + part

    @pl.when(i == 15)
    def _():
        hg = acc[...] * (1.0 / NN)
        out_ref[...] = jnp.dot(hg, wc_ref[...], preferred_element_type=jnp.float32,
                               precision=lax.Precision.HIGHEST) + bc_ref[...]


def _final_call(a0, a1, a2, a3, nd, b2r, Wc, bcr):
    return pl.pallas_call(
        _final_body,
        grid=(16,),
        in_specs=[pl.BlockSpec((RT, 32), lambda i: (i, 0)) for _ in range(4)] + [
            pl.BlockSpec((RT, 1), lambda i: (i, 0)),
            pl.BlockSpec((1, FH2), lambda i: (0, 0)),
            pl.BlockSpec((FH2, NCLS), lambda i: (0, 0)),
            pl.BlockSpec((1, NCLS), lambda i: (0, 0)),
        ],
        out_specs=pl.BlockSpec((1, NCLS), lambda i: (0, 0)),
        out_shape=jax.ShapeDtypeStruct((1, NCLS), jnp.float32),
        scratch_shapes=[pltpu.VMEM((1, FH2), jnp.float32)],
    )(a0, a1, a2, a3, nd, b2r, Wc, bcr)


def kernel(x, edge_index, W1, b1, W2, b2, Wc, bc):
    src = edge_index[0]
    dst = edge_index[1]
    pad = jnp.full((EP - EE,), NN, dtype=jnp.int32)
    sp = jnp.concatenate([src, pad])
    dp = jnp.concatenate([dst, pad])
    src_a = sp.reshape(16, NJF, CH)
    dst_a = dp.reshape(16, NJF, CH)
    src_c = sp.reshape(2, 16, NJH, CH)
    dst_c = dp.reshape(2, 16, NJH, CH)

    xpad = jnp.zeros((NP, 16), jnp.float32).at[:NN, :15].set(x)
    w1p = jnp.zeros((16, FH1), jnp.float32).at[:15].set(W1)
    z16 = jnp.zeros((RT, 16), jnp.float32)
    z32 = jnp.zeros((RT, 32), jnp.float32)
    o16 = jnp.ones((CH, 16), jnp.float32)

    degs = _deg_kernel()(src_a, dst_a, z16, o16)
    xn, ns, nd = _prep_call(degs, xpad)
    aggp = _agg1_kernel()(xn, src_c, dst_c, z16)
    t0, t1, t2, t3 = _dense_call(aggp, ns, nd, w1p, b1.reshape(1, FH1), W2)
    a0, a1, a2, a3 = _agg2_kernel()(t0, t1, t2, t3, src_a, dst_a, z32)
    return _final_call(a0, a1, a2, a3, nd, b2.reshape(1, FH2), Wc,
                       bc.reshape(1, NCLS))


# speedup vs baseline: 12.0149x; 1.4805x over previous
import functools

import jax
import jax.numpy as jnp
from jax import lax
from jax.experimental import pallas as pl
from jax.experimental.pallas import tpu as pltpu
from jax.experimental.pallas import tpu_sc as plsc

NN = 50000
NP = 50048
EE = 1600000
EP = 1605632
RT = 3128
CH = 128
NJF = 784
NJH = 392
FH1 = 256
FH2 = 128
NCLS = 10


@functools.lru_cache(maxsize=None)
def _deg_kernel():
    mesh = plsc.VectorSubcoreMesh(core_axis_name="c", subcore_axis_name="s")

    SB = 56

    @functools.partial(
        pl.kernel,
        out_type=jax.ShapeDtypeStruct((2, NP, 16), jnp.float32),
        mesh=mesh,
        compiler_params=pltpu.CompilerParams(use_tc_tiling_on_sc=False),
        scratch_types=[
            pltpu.VMEM((SB, CH), jnp.int32),
            pltpu.VMEM((CH, 16), jnp.float32),
            pltpu.VMEM_SHARED((NP, 16), jnp.float32),
        ],
    )
    def deg(src_hbm, dst_hbm, zrow_hbm, ones_hbm, out_hbm, idx_v, ones_v, acc_sh):
        c = lax.axis_index("c")
        s = lax.axis_index("s")
        row0 = s * RT
        pltpu.sync_copy(zrow_hbm, acc_sh.at[pl.ds(row0, RT)])
        pltpu.sync_copy(ones_hbm, ones_v)
        plsc.subcore_barrier()

        def blk(b, carry):
            @pl.when(c == 0)
            def _():
                pltpu.sync_copy(src_hbm.at[s, pl.ds(b * SB, SB)], idx_v)

            @pl.when(c == 1)
            def _():
                pltpu.sync_copy(dst_hbm.at[s, pl.ds(b * SB, SB)], idx_v)

            def body(j, carry2):
                pltpu.sync_copy(ones_v, acc_sh.at[idx_v.at[j]], add=True)
                return carry2

            return lax.fori_loop(0, SB, body, carry)

        lax.fori_loop(0, NJF // SB, blk, 0)
        plsc.subcore_barrier()
        pltpu.sync_copy(acc_sh.at[pl.ds(row0, RT)], out_hbm.at[c, pl.ds(row0, RT)])

    return deg


@functools.lru_cache(maxsize=None)
def _agg1_kernel():
    mesh = plsc.VectorSubcoreMesh(core_axis_name="c", subcore_axis_name="s")

    SB = 56

    @functools.partial(
        pl.kernel,
        out_type=jax.ShapeDtypeStruct((2, NP, 16), jnp.float32),
        mesh=mesh,
        compiler_params=pltpu.CompilerParams(use_tc_tiling_on_sc=False),
        scratch_types=[
            pltpu.VMEM((SB, CH), jnp.int32),
            pltpu.VMEM((SB, CH), jnp.int32),
            pltpu.VMEM((CH, 16), jnp.float32),
            pltpu.VMEM((CH, 16), jnp.float32),
            pltpu.VMEM((CH, 16), jnp.float32),
            pltpu.VMEM((CH, 16), jnp.float32),
            pltpu.VMEM_SHARED((NP, 16), jnp.float32),
            pltpu.SemaphoreType.DMA,
            pltpu.SemaphoreType.DMA,
            pltpu.SemaphoreType.DMA,
            pltpu.SemaphoreType.DMA,
        ],
    )
    def agg1(xn_hbm, src_hbm, dst_hbm, zrow_hbm, out_hbm,
             src_v, dst_v, rb0, rb1, rb2, rb3, acc_sh, sm0, sm1, sm2, sm3):
        c = lax.axis_index("c")
        s = lax.axis_index("s")
        row0 = s * RT
        rbs = (rb0, rb1, rb2, rb3)
        sms = (sm0, sm1, sm2, sm3)
        pltpu.sync_copy(zrow_hbm, acc_sh.at[pl.ds(row0, RT)])
        plsc.subcore_barrier()

        def blk(b, carry):
            pltpu.sync_copy(src_hbm.at[c, s, pl.ds(b * SB, SB)], src_v)
            pltpu.sync_copy(dst_hbm.at[c, s, pl.ds(b * SB, SB)], dst_v)
            for q in range(4):
                pltpu.async_copy(xn_hbm.at[src_v.at[q]], rbs[q], sms[q])

            def body(m, carry2):
                j = 4 * m
                for q in range(4):
                    pltpu.make_async_copy(
                        xn_hbm.at[src_v.at[j + q]], rbs[q], sms[q]).wait()
                    pltpu.sync_copy(rbs[q], acc_sh.at[dst_v.at[j + q]], add=True)

                    @pl.when(j + q + 4 < SB)
                    def _(j=j, q=q):
                        pltpu.async_copy(
                            xn_hbm.at[src_v.at[j + q + 4]], rbs[q], sms[q])
                return carry2

            return lax.fori_loop(0, SB // 4, body, carry)

        lax.fori_loop(0, NJH // SB, blk, 0)
        plsc.subcore_barrier()
        pltpu.sync_copy(acc_sh.at[pl.ds(row0, RT)], out_hbm.at[c, pl.ds(row0, RT)])

    return agg1


@functools.lru_cache(maxsize=None)
def _agg2_kernel():
    mesh = plsc.VectorSubcoreMesh(core_axis_name="c", subcore_axis_name="s")
    oshape = tuple(jax.ShapeDtypeStruct((NP, 32), jnp.float32) for _ in range(4))

    @functools.partial(
        pl.kernel,
        out_type=oshape,
        mesh=mesh,
        compiler_params=pltpu.CompilerParams(use_tc_tiling_on_sc=False),
        scratch_types=[
            pltpu.VMEM((16, CH), jnp.int32),
            pltpu.VMEM((16, CH), jnp.int32),
            pltpu.VMEM((CH, 32), jnp.float32),
            pltpu.VMEM((CH, 32), jnp.float32),
            pltpu.VMEM_SHARED((NP, 32), jnp.float32),
            pltpu.SemaphoreType.DMA,
            pltpu.SemaphoreType.DMA,
        ],
    )
    def agg2(t0, t1, t2, t3, src_hbm, dst_hbm, zrow_hbm,
             o0, o1, o2, o3, src_v, dst_v, rb0, rb1, acc_sh, sm0, sm1):
        SB = 16
        c = lax.axis_index("c")
        s = lax.axis_index("s")
        row0 = s * RT
        t_refs = (t0, t1, t2, t3)
        o_refs = (o0, o1, o2, o3)
        rbs = (rb0, rb1)
        sms = (sm0, sm1)
        for cc in range(2):
            @pl.when(c == cc)
            def _(cc=cc):
                for kk in range(2):
                    k = 2 * cc + kk
                    pltpu.sync_copy(zrow_hbm, acc_sh.at[pl.ds(row0, RT)])
                    plsc.subcore_barrier()

                    def blk(b, carry, k=k):
                        pltpu.sync_copy(src_hbm.at[s, pl.ds(b * SB, SB)], src_v)
                        pltpu.sync_copy(dst_hbm.at[s, pl.ds(b * SB, SB)], dst_v)
                        for q in range(2):
                            pltpu.async_copy(
                                t_refs[k].at[src_v.at[q]], rbs[q], sms[q])

                        def body(m, carry2, k=k):
                            j = 2 * m
                            for q in range(2):
                                pltpu.make_async_copy(
                                    t_refs[k].at[src_v.at[j + q]],
                                    rbs[q], sms[q]).wait()
                                pltpu.sync_copy(
                                    rbs[q], acc_sh.at[dst_v.at[j + q]], add=True)

                                @pl.when(j + q + 2 < SB)
                                def _(j=j, q=q, k=k):
                                    pltpu.async_copy(
                                        t_refs[k].at[src_v.at[j + q + 2]],
                                        rbs[q], sms[q])
                            return carry2

                        return lax.fori_loop(0, SB // 2, body, carry)

                    lax.fori_loop(0, NJF // SB, blk, 0)
                    plsc.subcore_barrier()
                    pltpu.sync_copy(acc_sh.at[pl.ds(row0, RT)],
                                    o_refs[k].at[pl.ds(row0, RT)])
                    plsc.subcore_barrier()

    return agg2


def _prep_body(degs_ref, x_ref, xn_ref, ns_ref, nd_ref):
    ns = lax.rsqrt(jnp.maximum(degs_ref[0][:, 0:1], 1.0))
    nd = lax.rsqrt(jnp.maximum(degs_ref[1][:, 0:1], 1.0))
    xn_ref[...] = x_ref[...] * ns
    ns_ref[...] = ns
    nd_ref[...] = nd


def _prep_call(degs, xpad):
    return pl.pallas_call(
        _prep_body,
        grid=(16,),
        in_specs=[
            pl.BlockSpec((2, RT, 16), lambda i: (0, i, 0)),
            pl.BlockSpec((RT, 16), lambda i: (i, 0)),
        ],
        out_specs=[
            pl.BlockSpec((RT, 16), lambda i: (i, 0)),
            pl.BlockSpec((RT, 1), lambda i: (i, 0)),
            pl.BlockSpec((RT, 1), lambda i: (i, 0)),
        ],
        out_shape=[
            jax.ShapeDtypeStruct((NP, 16), jnp.float32),
            jax.ShapeDtypeStruct((NP, 1), jnp.float32),
            jax.ShapeDtypeStruct((NP, 1), jnp.float32),
        ],
    )(degs, xpad)


def _dense_body(aggp_ref, ns_ref, nd_ref, w1_ref, b1_ref, w2_ref,
                t0_ref, t1_ref, t2_ref, t3_ref):
    agg = (aggp_ref[0] + aggp_ref[1]) * nd_ref[...]
    h1 = jnp.dot(agg, w1_ref[...], preferred_element_type=jnp.float32,
                 precision=lax.Precision.HIGHEST)
    h1 = jnp.maximum(h1 + b1_ref[...], 0.0)
    h1n = h1 * ns_ref[...]
    t = jnp.dot(h1n, w2_ref[...], preferred_element_type=jnp.float32,
                precision=lax.Precision.HIGHEST)
    t0_ref[...] = t[:, 0:32]
    t1_ref[...] = t[:, 32:64]
    t2_ref[...] = t[:, 64:96]
    t3_ref[...] = t[:, 96:128]


def _dense_call(aggp, ns, nd, w1p, b1r, W2):
    return pl.pallas_call(
        _dense_body,
        grid=(16,),
        in_specs=[
            pl.BlockSpec((2, RT, 16), lambda i: (0, i, 0)),
            pl.BlockSpec((RT, 1), lambda i: (i, 0)),
            pl.BlockSpec((RT, 1), lambda i: (i, 0)),
            pl.BlockSpec((16, FH1), lambda i: (0, 0)),
            pl.BlockSpec((1, FH1), lambda i: (0, 0)),
            pl.BlockSpec((FH1, FH2), lambda i: (0, 0)),
        ],
        out_specs=[pl.BlockSpec((RT, 32), lambda i: (i, 0)) for _ in range(4)],
        out_shape=[jax.ShapeDtypeStruct((NP, 32), jnp.float32) for _ in range(4)],
    )(aggp, ns, nd, w1p, b1r, W2)


def _final_body(a0, a1, a2, a3, nd_ref, b2_ref, wc_ref, bc_ref, out_ref, acc):
    i = pl.program_id(0)
    h = jnp.concatenate([a0[...], a1[...], a2[...], a3[...]], axis=1)
    h2 = jnp.maximum(h * nd_ref[...] + b2_ref[...], 0.0)
    rows = RT * i + lax.broadcasted_iota(jnp.int32, (RT, 1), 0)
    h2 = jnp.where(rows < NN, h2, 0.0)
    part = jnp.sum(h2, axis=0, keepdims=True)

    @pl.when(i == 0)
    def _():
        acc[...] = part

    @pl.when(i > 0)
    def _():
        acc[...] = acc[...] + part

    @pl.when(i == 15)
    def _():
        hg = acc[...] * (1.0 / NN)
        out_ref[...] = jnp.dot(hg, wc_ref[...], preferred_element_type=jnp.float32,
                               precision=lax.Precision.HIGHEST) + bc_ref[...]


def _final_call(a0, a1, a2, a3, nd, b2r, Wc, bcr):
    return pl.pallas_call(
        _final_body,
        grid=(16,),
        in_specs=[pl.BlockSpec((RT, 32), lambda i: (i, 0)) for _ in range(4)] + [
            pl.BlockSpec((RT, 1), lambda i: (i, 0)),
            pl.BlockSpec((1, FH2), lambda i: (0, 0)),
            pl.BlockSpec((FH2, NCLS), lambda i: (0, 0)),
            pl.BlockSpec((1, NCLS), lambda i: (0, 0)),
        ],
        out_specs=pl.BlockSpec((1, NCLS), lambda i: (0, 0)),
        out_shape=jax.ShapeDtypeStruct((1, NCLS), jnp.float32),
        scratch_shapes=[pltpu.VMEM((1, FH2), jnp.float32)],
    )(a0, a1, a2, a3, nd, b2r, Wc, bcr)


def kernel(x, edge_index, W1, b1, W2, b2, Wc, bc):
    src = edge_index[0]
    dst = edge_index[1]
    pad = jnp.full((EP - EE,), NN, dtype=jnp.int32)
    sp = jnp.concatenate([src, pad])
    dp = jnp.concatenate([dst, pad])
    src_a = sp.reshape(16, NJF, CH)
    dst_a = dp.reshape(16, NJF, CH)
    src_c = sp.reshape(2, 16, NJH, CH)
    dst_c = dp.reshape(2, 16, NJH, CH)

    xpad = jnp.zeros((NP, 16), jnp.float32).at[:NN, :15].set(x)
    w1p = jnp.zeros((16, FH1), jnp.float32).at[:15].set(W1)
    z16 = jnp.zeros((RT, 16), jnp.float32)
    z32 = jnp.zeros((RT, 32), jnp.float32)
    o16 = jnp.ones((CH, 16), jnp.float32)

    degs = _deg_kernel()(src_a, dst_a, z16, o16)
    xn, ns, nd = _prep_call(degs, xpad)
    aggp = _agg1_kernel()(xn, src_c, dst_c, z16)
    t0, t1, t2, t3 = _dense_call(aggp, ns, nd, w1p, b1.reshape(1, FH1), W2)
    a0, a1, a2, a3 = _agg2_kernel()(t0, t1, t2, t3, src_a, dst_a, z32)
    return _final_call(a0, a1, a2, a3, nd, b2.reshape(1, FH2), Wc,
                       bc.reshape(1, NCLS))


# speedup vs baseline: 12.4022x; 1.0322x over previous
import functools

import jax
import jax.numpy as jnp
from jax import lax
from jax.experimental import pallas as pl
from jax.experimental.pallas import tpu as pltpu
from jax.experimental.pallas import tpu_sc as plsc

NN = 50000
NP = 50048
EE = 1600000
EP = 1605632
RT = 3128
CH = 128
NJF = 784
NJH = 392
FH1 = 256
FH2 = 128
NCLS = 10


CH2 = 256
SB2 = 14
NB2F = 28
NB2H = 14


@functools.lru_cache(maxsize=None)
def _deg_kernel():
    mesh = plsc.VectorSubcoreMesh(core_axis_name="c", subcore_axis_name="s")

    @functools.partial(
        pl.kernel,
        out_type=jax.ShapeDtypeStruct((2, NP, 16), jnp.float32),
        mesh=mesh,
        compiler_params=pltpu.CompilerParams(use_tc_tiling_on_sc=False),
        scratch_types=[
            pltpu.VMEM((SB2, CH2), jnp.int32),
            pltpu.VMEM((CH2, 16), jnp.float32),
            pltpu.VMEM_SHARED((NP, 16), jnp.float32),
            pltpu.SemaphoreType.DMA,
        ],
    )
    def deg(src_hbm, dst_hbm, zrow_hbm, ones_hbm, out_hbm, idx_v, ones_v,
            acc_sh, ssm):
        c = lax.axis_index("c")
        s = lax.axis_index("s")
        row0 = s * RT
        pltpu.sync_copy(zrow_hbm, acc_sh.at[pl.ds(row0, RT)])
        pltpu.sync_copy(ones_hbm, ones_v)
        plsc.subcore_barrier()

        def blk(b, carry):
            g = s * NB2F + b

            @pl.when(c == 0)
            def _():
                pltpu.sync_copy(src_hbm.at[g], idx_v)

            @pl.when(c == 1)
            def _():
                pltpu.sync_copy(dst_hbm.at[g], idx_v)

            def body(j, carry2):
                pltpu.async_copy(ones_v, acc_sh.at[idx_v.at[j]], ssm, add=True)

                @pl.when(jnp.logical_or(j > 0, b > 0))
                def _():
                    pltpu.make_async_copy(
                        ones_v, acc_sh.at[idx_v.at[j]], ssm).wait()
                return carry2

            return lax.fori_loop(0, SB2, body, carry)

        lax.fori_loop(0, NB2F, blk, 0)
        pltpu.make_async_copy(ones_v, acc_sh.at[idx_v.at[0]], ssm).wait()
        plsc.subcore_barrier()
        pltpu.sync_copy(acc_sh.at[pl.ds(row0, RT)], out_hbm.at[c, pl.ds(row0, RT)])

    return deg


@functools.lru_cache(maxsize=None)
def _agg1_kernel():
    mesh = plsc.VectorSubcoreMesh(core_axis_name="c", subcore_axis_name="s")
    NB = 7

    @functools.partial(
        pl.kernel,
        out_type=jax.ShapeDtypeStruct((2, NP, 16), jnp.float32),
        mesh=mesh,
        compiler_params=pltpu.CompilerParams(use_tc_tiling_on_sc=False),
        scratch_types=[
            pltpu.VMEM((SB2, CH2), jnp.int32),
            pltpu.VMEM((SB2, CH2), jnp.int32),
            [pltpu.VMEM((CH2, 16), jnp.float32) for _ in range(NB)],
            pltpu.VMEM_SHARED((NP, 16), jnp.float32),
            [pltpu.SemaphoreType.DMA for _ in range(NB)],
            [pltpu.SemaphoreType.DMA for _ in range(NB)],
        ],
    )
    def agg1(xn_hbm, src_hbm, dst_hbm, zrow_hbm, out_hbm,
             src_v, dst_v, rbs, acc_sh, gsm, ssm):
        c = lax.axis_index("c")
        s = lax.axis_index("s")
        row0 = s * RT
        pltpu.sync_copy(zrow_hbm, acc_sh.at[pl.ds(row0, RT)])
        plsc.subcore_barrier()

        def blk(b, carry):
            g = (c * 16 + s) * NB2H + b
            pltpu.sync_copy(src_hbm.at[g], src_v)
            pltpu.sync_copy(dst_hbm.at[g], dst_v)
            for q in range(NB):
                pltpu.async_copy(xn_hbm.at[src_v.at[q]], rbs[q], gsm[q])

            def body(m, carry2):
                for q in range(NB):
                    j = NB * m + q
                    pltpu.make_async_copy(
                        xn_hbm.at[src_v.at[j]], rbs[q], gsm[q]).wait()
                    pltpu.async_copy(
                        rbs[q], acc_sh.at[dst_v.at[j]], ssm[q], add=True)
                    qp = (q - 1) % NB

                    @pl.when(jnp.logical_and(j >= 1, j + NB - 1 < SB2))
                    def _(j=j, q=q, qp=qp):
                        pltpu.make_async_copy(
                            rbs[qp], acc_sh.at[dst_v.at[j]], ssm[qp]).wait()
                        pltpu.async_copy(
                            xn_hbm.at[src_v.at[j + NB - 1]], rbs[qp], gsm[qp])
                return carry2

            lax.fori_loop(0, SB2 // NB, body, carry)
            for q in range(NB):
                pltpu.make_async_copy(
                    rbs[q], acc_sh.at[dst_v.at[q]], ssm[q]).wait()
            return carry

        lax.fori_loop(0, NB2H, blk, 0)
        plsc.subcore_barrier()
        pltpu.sync_copy(acc_sh.at[pl.ds(row0, RT)], out_hbm.at[c, pl.ds(row0, RT)])

    return agg1


@functools.lru_cache(maxsize=None)
def _agg2_kernel():
    mesh = plsc.VectorSubcoreMesh(core_axis_name="c", subcore_axis_name="s")
    oshape = tuple(jax.ShapeDtypeStruct((NP, 32), jnp.float32) for _ in range(4))

    @functools.partial(
        pl.kernel,
        out_type=oshape,
        mesh=mesh,
        compiler_params=pltpu.CompilerParams(use_tc_tiling_on_sc=False),
        scratch_types=[
            pltpu.VMEM((16, CH), jnp.int32),
            pltpu.VMEM((16, CH), jnp.int32),
            pltpu.VMEM((CH, 32), jnp.float32),
            pltpu.VMEM((CH, 32), jnp.float32),
            pltpu.VMEM_SHARED((NP, 32), jnp.float32),
            pltpu.SemaphoreType.DMA,
            pltpu.SemaphoreType.DMA,
        ],
    )
    def agg2(t0, t1, t2, t3, src_hbm, dst_hbm, zrow_hbm,
             o0, o1, o2, o3, src_v, dst_v, rb0, rb1, acc_sh, sm0, sm1):
        SB = 16
        c = lax.axis_index("c")
        s = lax.axis_index("s")
        row0 = s * RT
        t_refs = (t0, t1, t2, t3)
        o_refs = (o0, o1, o2, o3)
        rbs = (rb0, rb1)
        sms = (sm0, sm1)
        for cc in range(2):
            @pl.when(c == cc)
            def _(cc=cc):
                for kk in range(2):
                    k = 2 * cc + kk
                    pltpu.sync_copy(zrow_hbm, acc_sh.at[pl.ds(row0, RT)])
                    plsc.subcore_barrier()

                    def blk(b, carry, k=k):
                        pltpu.sync_copy(src_hbm.at[s, pl.ds(b * SB, SB)], src_v)
                        pltpu.sync_copy(dst_hbm.at[s, pl.ds(b * SB, SB)], dst_v)
                        for q in range(2):
                            pltpu.async_copy(
                                t_refs[k].at[src_v.at[q]], rbs[q], sms[q])

                        def body(m, carry2, k=k):
                            j = 2 * m
                            for q in range(2):
                                pltpu.make_async_copy(
                                    t_refs[k].at[src_v.at[j + q]],
                                    rbs[q], sms[q]).wait()
                                pltpu.sync_copy(
                                    rbs[q], acc_sh.at[dst_v.at[j + q]], add=True)

                                @pl.when(j + q + 2 < SB)
                                def _(j=j, q=q, k=k):
                                    pltpu.async_copy(
                                        t_refs[k].at[src_v.at[j + q + 2]],
                                        rbs[q], sms[q])
                            return carry2

                        return lax.fori_loop(0, SB // 2, body, carry)

                    lax.fori_loop(0, NJF // SB, blk, 0)
                    plsc.subcore_barrier()
                    pltpu.sync_copy(acc_sh.at[pl.ds(row0, RT)],
                                    o_refs[k].at[pl.ds(row0, RT)])
                    plsc.subcore_barrier()

    return agg2


def _prep_body(degs_ref, x_ref, xn_ref, ns_ref, nd_ref):
    ns = lax.rsqrt(jnp.maximum(degs_ref[0][:, 0:1], 1.0))
    nd = lax.rsqrt(jnp.maximum(degs_ref[1][:, 0:1], 1.0))
    xn_ref[...] = x_ref[...] * ns
    ns_ref[...] = ns
    nd_ref[...] = nd


def _prep_call(degs, xpad):
    return pl.pallas_call(
        _prep_body,
        grid=(16,),
        in_specs=[
            pl.BlockSpec((2, RT, 16), lambda i: (0, i, 0)),
            pl.BlockSpec((RT, 16), lambda i: (i, 0)),
        ],
        out_specs=[
            pl.BlockSpec((RT, 16), lambda i: (i, 0)),
            pl.BlockSpec((RT, 1), lambda i: (i, 0)),
            pl.BlockSpec((RT, 1), lambda i: (i, 0)),
        ],
        out_shape=[
            jax.ShapeDtypeStruct((NP, 16), jnp.float32),
            jax.ShapeDtypeStruct((NP, 1), jnp.float32),
            jax.ShapeDtypeStruct((NP, 1), jnp.float32),
        ],
    )(degs, xpad)


def _dense_body(aggp_ref, ns_ref, nd_ref, w1_ref, b1_ref, w2_ref,
                t0_ref, t1_ref, t2_ref, t3_ref):
    agg = (aggp_ref[0] + aggp_ref[1]) * nd_ref[...]
    h1 = jnp.dot(agg, w1_ref[...], preferred_element_type=jnp.float32,
                 precision=lax.Precision.HIGHEST)
    h1 = jnp.maximum(h1 + b1_ref[...], 0.0)
    h1n = h1 * ns_ref[...]
    t = jnp.dot(h1n, w2_ref[...], preferred_element_type=jnp.float32,
                precision=lax.Precision.HIGHEST)
    t0_ref[...] = t[:, 0:32]
    t1_ref[...] = t[:, 32:64]
    t2_ref[...] = t[:, 64:96]
    t3_ref[...] = t[:, 96:128]


def _dense_call(aggp, ns, nd, w1p, b1r, W2):
    return pl.pallas_call(
        _dense_body,
        grid=(16,),
        in_specs=[
            pl.BlockSpec((2, RT, 16), lambda i: (0, i, 0)),
            pl.BlockSpec((RT, 1), lambda i: (i, 0)),
            pl.BlockSpec((RT, 1), lambda i: (i, 0)),
            pl.BlockSpec((16, FH1), lambda i: (0, 0)),
            pl.BlockSpec((1, FH1), lambda i: (0, 0)),
            pl.BlockSpec((FH1, FH2), lambda i: (0, 0)),
        ],
        out_specs=[pl.BlockSpec((RT, 32), lambda i: (i, 0)) for _ in range(4)],
        out_shape=[jax.ShapeDtypeStruct((NP, 32), jnp.float32) for _ in range(4)],
    )(aggp, ns, nd, w1p, b1r, W2)


def _final_body(a0, a1, a2, a3, nd_ref, b2_ref, wc_ref, bc_ref, out_ref, acc):
    i = pl.program_id(0)
    h = jnp.concatenate([a0[...], a1[...], a2[...], a3[...]], axis=1)
    h2 = jnp.maximum(h * nd_ref[...] + b2_ref[...], 0.0)
    rows = RT * i + lax.broadcasted_iota(jnp.int32, (RT, 1), 0)
    h2 = jnp.where(rows < NN, h2, 0.0)
    part = jnp.sum(h2, axis=0, keepdims=True)

    @pl.when(i == 0)
    def _():
        acc[...] = part

    @pl.when(i > 0)
    def _():
        acc[...] = acc[...] + part

    @pl.when(i == 15)
    def _():
        hg = acc[...] * (1.0 / NN)
        out_ref[...] = jnp.dot(hg, wc_ref[...], preferred_element_type=jnp.float32,
                               precision=lax.Precision.HIGHEST) + bc_ref[...]


def _final_call(a0, a1, a2, a3, nd, b2r, Wc, bcr):
    return pl.pallas_call(
        _final_body,
        grid=(16,),
        in_specs=[pl.BlockSpec((RT, 32), lambda i: (i, 0)) for _ in range(4)] + [
            pl.BlockSpec((RT, 1), lambda i: (i, 0)),
            pl.BlockSpec((1, FH2), lambda i: (0, 0)),
            pl.BlockSpec((FH2, NCLS), lambda i: (0, 0)),
            pl.BlockSpec((1, NCLS), lambda i: (0, 0)),
        ],
        out_specs=pl.BlockSpec((1, NCLS), lambda i: (0, 0)),
        out_shape=jax.ShapeDtypeStruct((1, NCLS), jnp.float32),
        scratch_shapes=[pltpu.VMEM((1, FH2), jnp.float32)],
    )(a0, a1, a2, a3, nd, b2r, Wc, bcr)


def kernel(x, edge_index, W1, b1, W2, b2, Wc, bc):
    src = edge_index[0]
    dst = edge_index[1]
    pad = jnp.full((EP - EE,), NN, dtype=jnp.int32)
    sp = jnp.concatenate([src, pad])
    dp = jnp.concatenate([dst, pad])
    src_a = sp.reshape(16, NJF, CH)
    dst_a = dp.reshape(16, NJF, CH)
    src_w = sp.reshape(EP // (SB2 * CH2), SB2, CH2)
    dst_w = dp.reshape(EP // (SB2 * CH2), SB2, CH2)

    xpad = jnp.zeros((NP, 16), jnp.float32).at[:NN, :15].set(x)
    w1p = jnp.zeros((16, FH1), jnp.float32).at[:15].set(W1)
    z16 = jnp.zeros((RT, 16), jnp.float32)
    z32 = jnp.zeros((RT, 32), jnp.float32)
    o16 = jnp.ones((CH2, 16), jnp.float32)

    degs = _deg_kernel()(src_w, dst_w, z16, o16)
    xn, ns, nd = _prep_call(degs, xpad)
    aggp = _agg1_kernel()(xn, src_w, dst_w, z16)
    t0, t1, t2, t3 = _dense_call(aggp, ns, nd, w1p, b1.reshape(1, FH1), W2)
    a0, a1, a2, a3 = _agg2_kernel()(t0, t1, t2, t3, src_a, dst_a, z32)
    return _final_call(a0, a1, a2, a3, nd, b2.reshape(1, FH2), Wc,
                       bc.reshape(1, NCLS))


# speedup vs baseline: 12.9732x; 1.0460x over previous
import functools

import jax
import jax.numpy as jnp
from jax import lax
from jax.experimental import pallas as pl
from jax.experimental.pallas import tpu as pltpu
from jax.experimental.pallas import tpu_sc as plsc

NN = 50000
NP = 50048
EE = 1600000
EP = 1605632
RT = 3128
CH = 128
NJF = 784
NJH = 392
FH1 = 256
FH2 = 128
NCLS = 10


CH2 = 256
SB2 = 14
NB2F = 28
NB2H = 14


@functools.lru_cache(maxsize=None)
def _deg_kernel():
    mesh = plsc.VectorSubcoreMesh(core_axis_name="c", subcore_axis_name="s")

    @functools.partial(
        pl.kernel,
        out_type=jax.ShapeDtypeStruct((2, NP, 16), jnp.float32),
        mesh=mesh,
        compiler_params=pltpu.CompilerParams(use_tc_tiling_on_sc=False),
        scratch_types=[
            pltpu.VMEM((SB2, CH2), jnp.int32),
            pltpu.VMEM((CH2, 16), jnp.float32),
            pltpu.VMEM_SHARED((NP, 16), jnp.float32),
            pltpu.SemaphoreType.DMA,
        ],
    )
    def deg(src_hbm, dst_hbm, zrow_hbm, ones_hbm, out_hbm, idx_v, ones_v,
            acc_sh, ssm):
        c = lax.axis_index("c")
        s = lax.axis_index("s")
        row0 = s * RT
        pltpu.sync_copy(zrow_hbm, acc_sh.at[pl.ds(row0, RT)])
        pltpu.sync_copy(ones_hbm, ones_v)
        plsc.subcore_barrier()

        def blk(b, carry):
            g = s * NB2F + b

            @pl.when(c == 0)
            def _():
                pltpu.sync_copy(src_hbm.at[g], idx_v)

            @pl.when(c == 1)
            def _():
                pltpu.sync_copy(dst_hbm.at[g], idx_v)

            def body(j, carry2):
                pltpu.async_copy(ones_v, acc_sh.at[idx_v.at[j]], ssm, add=True)

                @pl.when(jnp.logical_or(j > 0, b > 0))
                def _():
                    pltpu.make_async_copy(
                        ones_v, acc_sh.at[idx_v.at[j]], ssm).wait()
                return carry2

            return lax.fori_loop(0, SB2, body, carry)

        lax.fori_loop(0, NB2F, blk, 0)
        pltpu.make_async_copy(ones_v, acc_sh.at[idx_v.at[0]], ssm).wait()
        plsc.subcore_barrier()
        pltpu.sync_copy(acc_sh.at[pl.ds(row0, RT)], out_hbm.at[c, pl.ds(row0, RT)])

    return deg


@functools.lru_cache(maxsize=None)
def _agg1_kernel():
    mesh = plsc.VectorSubcoreMesh(core_axis_name="c", subcore_axis_name="s")
    NB = 7

    @functools.partial(
        pl.kernel,
        out_type=jax.ShapeDtypeStruct((2, NP, 16), jnp.float32),
        mesh=mesh,
        compiler_params=pltpu.CompilerParams(use_tc_tiling_on_sc=False),
        scratch_types=[
            pltpu.VMEM((SB2, CH2), jnp.int32),
            pltpu.VMEM((SB2, CH2), jnp.int32),
            [pltpu.VMEM((CH2, 16), jnp.float32) for _ in range(NB)],
            pltpu.VMEM_SHARED((NP, 16), jnp.float32),
            [pltpu.SemaphoreType.DMA for _ in range(NB)],
            [pltpu.SemaphoreType.DMA for _ in range(NB)],
        ],
    )
    def agg1(xn_hbm, src_hbm, dst_hbm, zrow_hbm, out_hbm,
             src_v, dst_v, rbs, acc_sh, gsm, ssm):
        c = lax.axis_index("c")
        s = lax.axis_index("s")
        row0 = s * RT
        pltpu.sync_copy(zrow_hbm, acc_sh.at[pl.ds(row0, RT)])
        plsc.subcore_barrier()

        def blk(b, carry):
            g = (c * 16 + s) * NB2H + b
            pltpu.sync_copy(src_hbm.at[g], src_v)
            pltpu.sync_copy(dst_hbm.at[g], dst_v)
            for q in range(NB):
                pltpu.async_copy(xn_hbm.at[src_v.at[q]], rbs[q], gsm[q])

            def body(m, carry2):
                for q in range(NB):
                    j = NB * m + q
                    pltpu.make_async_copy(
                        xn_hbm.at[src_v.at[j]], rbs[q], gsm[q]).wait()
                    pltpu.async_copy(
                        rbs[q], acc_sh.at[dst_v.at[j]], ssm[q], add=True)
                    qp = (q - 1) % NB

                    @pl.when(jnp.logical_and(j >= 1, j + NB - 1 < SB2))
                    def _(j=j, q=q, qp=qp):
                        pltpu.make_async_copy(
                            rbs[qp], acc_sh.at[dst_v.at[j]], ssm[qp]).wait()
                        pltpu.async_copy(
                            xn_hbm.at[src_v.at[j + NB - 1]], rbs[qp], gsm[qp])
                return carry2

            lax.fori_loop(0, SB2 // NB, body, carry)
            for q in range(NB):
                pltpu.make_async_copy(
                    rbs[q], acc_sh.at[dst_v.at[q]], ssm[q]).wait()
            return carry

        lax.fori_loop(0, NB2H, blk, 0)
        plsc.subcore_barrier()
        pltpu.sync_copy(acc_sh.at[pl.ds(row0, RT)], out_hbm.at[c, pl.ds(row0, RT)])

    return agg1


@functools.lru_cache(maxsize=None)
def _agg2_kernel():
    mesh = plsc.VectorSubcoreMesh(core_axis_name="c", subcore_axis_name="s")
    NB = 7
    oshape = tuple(jax.ShapeDtypeStruct((NP, 16), jnp.float32) for _ in range(8))

    @functools.partial(
        pl.kernel,
        out_type=oshape,
        mesh=mesh,
        compiler_params=pltpu.CompilerParams(use_tc_tiling_on_sc=False),
        scratch_types=[
            pltpu.VMEM((SB2, CH2), jnp.int32),
            pltpu.VMEM((SB2, CH2), jnp.int32),
            [pltpu.VMEM((CH2, 16), jnp.float32) for _ in range(NB)],
            pltpu.VMEM_SHARED((NP, 16), jnp.float32),
            [pltpu.SemaphoreType.DMA for _ in range(NB)],
            [pltpu.SemaphoreType.DMA for _ in range(NB)],
        ],
    )
    def agg2(t0, t1, t2, t3, t4, t5, t6, t7, src_hbm, dst_hbm, zrow_hbm,
             o0, o1, o2, o3, o4, o5, o6, o7,
             src_v, dst_v, rbs, acc_sh, gsm, ssm):
        c = lax.axis_index("c")
        s = lax.axis_index("s")
        row0 = s * RT
        t_refs = (t0, t1, t2, t3, t4, t5, t6, t7)
        o_refs = (o0, o1, o2, o3, o4, o5, o6, o7)
        for cc in range(2):
            @pl.when(c == cc)
            def _(cc=cc):
                for kk in range(4):
                    k = 4 * cc + kk
                    tbl = t_refs[k]
                    pltpu.sync_copy(zrow_hbm, acc_sh.at[pl.ds(row0, RT)])
                    plsc.subcore_barrier()

                    def blk(b, carry, tbl=tbl):
                        g = s * NB2F + b
                        pltpu.sync_copy(src_hbm.at[g], src_v)
                        pltpu.sync_copy(dst_hbm.at[g], dst_v)
                        for q in range(NB):
                            pltpu.async_copy(tbl.at[src_v.at[q]], rbs[q], gsm[q])

                        def body(m, carry2, tbl=tbl):
                            for q in range(NB):
                                j = NB * m + q
                                pltpu.make_async_copy(
                                    tbl.at[src_v.at[j]], rbs[q], gsm[q]).wait()
                                pltpu.async_copy(
                                    rbs[q], acc_sh.at[dst_v.at[j]], ssm[q],
                                    add=True)
                                qp = (q - 1) % NB

                                @pl.when(jnp.logical_and(j >= 1,
                                                         j + NB - 1 < SB2))
                                def _(j=j, qp=qp, tbl=tbl):
                                    pltpu.make_async_copy(
                                        rbs[qp], acc_sh.at[dst_v.at[j]],
                                        ssm[qp]).wait()
                                    pltpu.async_copy(
                                        tbl.at[src_v.at[j + NB - 1]],
                                        rbs[qp], gsm[qp])
                            return carry2

                        lax.fori_loop(0, SB2 // NB, body, carry)
                        for q in range(NB):
                            pltpu.make_async_copy(
                                rbs[q], acc_sh.at[dst_v.at[q]], ssm[q]).wait()
                        return carry

                    lax.fori_loop(0, NB2F, blk, 0)
                    plsc.subcore_barrier()
                    pltpu.sync_copy(acc_sh.at[pl.ds(row0, RT)],
                                    o_refs[k].at[pl.ds(row0, RT)])
                    plsc.subcore_barrier()

    return agg2


def _prep_body(degs_ref, x_ref, xn_ref, ns_ref, nd_ref):
    ns = lax.rsqrt(jnp.maximum(degs_ref[0][:, 0:1], 1.0))
    nd = lax.rsqrt(jnp.maximum(degs_ref[1][:, 0:1], 1.0))
    xn_ref[...] = x_ref[...] * ns
    ns_ref[...] = ns
    nd_ref[...] = nd


def _prep_call(degs, xpad):
    return pl.pallas_call(
        _prep_body,
        grid=(16,),
        in_specs=[
            pl.BlockSpec((2, RT, 16), lambda i: (0, i, 0)),
            pl.BlockSpec((RT, 16), lambda i: (i, 0)),
        ],
        out_specs=[
            pl.BlockSpec((RT, 16), lambda i: (i, 0)),
            pl.BlockSpec((RT, 1), lambda i: (i, 0)),
            pl.BlockSpec((RT, 1), lambda i: (i, 0)),
        ],
        out_shape=[
            jax.ShapeDtypeStruct((NP, 16), jnp.float32),
            jax.ShapeDtypeStruct((NP, 1), jnp.float32),
            jax.ShapeDtypeStruct((NP, 1), jnp.float32),
        ],
    )(degs, xpad)


def _dense_body(aggp_ref, ns_ref, nd_ref, w1_ref, b1_ref, w2_ref, *t_refs):
    agg = (aggp_ref[0] + aggp_ref[1]) * nd_ref[...]
    h1 = jnp.dot(agg, w1_ref[...], preferred_element_type=jnp.float32,
                 precision=lax.Precision.HIGHEST)
    h1 = jnp.maximum(h1 + b1_ref[...], 0.0)
    h1n = h1 * ns_ref[...]
    t = jnp.dot(h1n, w2_ref[...], preferred_element_type=jnp.float32,
                precision=lax.Precision.HIGHEST)
    for k in range(8):
        t_refs[k][...] = t[:, 16 * k:16 * k + 16]


def _dense_call(aggp, ns, nd, w1p, b1r, W2):
    return pl.pallas_call(
        _dense_body,
        grid=(16,),
        in_specs=[
            pl.BlockSpec((2, RT, 16), lambda i: (0, i, 0)),
            pl.BlockSpec((RT, 1), lambda i: (i, 0)),
            pl.BlockSpec((RT, 1), lambda i: (i, 0)),
            pl.BlockSpec((16, FH1), lambda i: (0, 0)),
            pl.BlockSpec((1, FH1), lambda i: (0, 0)),
            pl.BlockSpec((FH1, FH2), lambda i: (0, 0)),
        ],
        out_specs=[pl.BlockSpec((RT, 16), lambda i: (i, 0)) for _ in range(8)],
        out_shape=[jax.ShapeDtypeStruct((NP, 16), jnp.float32) for _ in range(8)],
    )(aggp, ns, nd, w1p, b1r, W2)


def _final_body(*refs):
    a_refs = refs[:8]
    nd_ref, b2_ref, wc_ref, bc_ref, out_ref, acc = refs[8:]
    i = pl.program_id(0)
    h = jnp.concatenate([a[...] for a in a_refs], axis=1)
    h2 = jnp.maximum(h * nd_ref[...] + b2_ref[...], 0.0)
    rows = RT * i + lax.broadcasted_iota(jnp.int32, (RT, 1), 0)
    h2 = jnp.where(rows < NN, h2, 0.0)
    part = jnp.sum(h2, axis=0, keepdims=True)

    @pl.when(i == 0)
    def _():
        acc[...] = part

    @pl.when(i > 0)
    def _():
        acc[...] = acc[...] + part

    @pl.when(i == 15)
    def _():
        hg = acc[...] * (1.0 / NN)
        out_ref[...] = jnp.dot(hg, wc_ref[...], preferred_element_type=jnp.float32,
                               precision=lax.Precision.HIGHEST) + bc_ref[...]


def _final_call(aggs, nd, b2r, Wc, bcr):
    return pl.pallas_call(
        _final_body,
        grid=(16,),
        in_specs=[pl.BlockSpec((RT, 16), lambda i: (i, 0)) for _ in range(8)] + [
            pl.BlockSpec((RT, 1), lambda i: (i, 0)),
            pl.BlockSpec((1, FH2), lambda i: (0, 0)),
            pl.BlockSpec((FH2, NCLS), lambda i: (0, 0)),
            pl.BlockSpec((1, NCLS), lambda i: (0, 0)),
        ],
        out_specs=pl.BlockSpec((1, NCLS), lambda i: (0, 0)),
        out_shape=jax.ShapeDtypeStruct((1, NCLS), jnp.float32),
        scratch_shapes=[pltpu.VMEM((1, FH2), jnp.float32)],
    )(*aggs, nd, b2r, Wc, bcr)


def kernel(x, edge_index, W1, b1, W2, b2, Wc, bc):
    src = edge_index[0]
    dst = edge_index[1]
    pad = jnp.full((EP - EE,), NN, dtype=jnp.int32)
    sp = jnp.concatenate([src, pad])
    dp = jnp.concatenate([dst, pad])
    src_w = sp.reshape(EP // (SB2 * CH2), SB2, CH2)
    dst_w = dp.reshape(EP // (SB2 * CH2), SB2, CH2)

    xpad = jnp.zeros((NP, 16), jnp.float32).at[:NN, :15].set(x)
    w1p = jnp.zeros((16, FH1), jnp.float32).at[:15].set(W1)
    z16 = jnp.zeros((RT, 16), jnp.float32)
    o16 = jnp.ones((CH2, 16), jnp.float32)

    degs = _deg_kernel()(src_w, dst_w, z16, o16)
    xn, ns, nd = _prep_call(degs, xpad)
    aggp = _agg1_kernel()(xn, src_w, dst_w, z16)
    ts = _dense_call(aggp, ns, nd, w1p, b1.reshape(1, FH1), W2)
    aggs = _agg2_kernel()(*ts, src_w, dst_w, z16)
    return _final_call(aggs, nd, b2.reshape(1, FH2), Wc,
                       bc.reshape(1, NCLS))


# speedup vs baseline: 13.1647x; 1.0148x over previous
import functools

import jax
import jax.numpy as jnp
from jax import lax
from jax.experimental import pallas as pl
from jax.experimental.pallas import tpu as pltpu
from jax.experimental.pallas import tpu_sc as plsc

NN = 50000
NP = 50048
EE = 1600000
EP = 1605632
RT = 3128
CH = 128
NJF = 784
NJH = 392
FH1 = 256
FH2 = 128
NCLS = 10


CH2 = 256
SB2 = 14
NB2F = 28
NB2H = 14


@functools.lru_cache(maxsize=None)
def _deg_kernel():
    mesh = plsc.VectorSubcoreMesh(core_axis_name="c", subcore_axis_name="s")

    @functools.partial(
        pl.kernel,
        out_type=jax.ShapeDtypeStruct((2, NP, 16), jnp.float32),
        mesh=mesh,
        compiler_params=pltpu.CompilerParams(use_tc_tiling_on_sc=False),
        scratch_types=[
            pltpu.VMEM((SB2, CH2), jnp.int32),
            pltpu.VMEM((CH2, 16), jnp.float32),
            pltpu.VMEM_SHARED((NP, 16), jnp.float32),
            pltpu.SemaphoreType.DMA,
        ],
    )
    def deg(src_hbm, dst_hbm, zrow_hbm, ones_hbm, out_hbm, idx_v, ones_v,
            acc_sh, ssm):
        c = lax.axis_index("c")
        s = lax.axis_index("s")
        row0 = s * RT
        pltpu.sync_copy(zrow_hbm, acc_sh.at[pl.ds(row0, RT)])
        pltpu.sync_copy(ones_hbm, ones_v)
        plsc.subcore_barrier()

        def blk(b, carry):
            g = s * NB2F + b

            @pl.when(c == 0)
            def _():
                pltpu.sync_copy(src_hbm.at[g], idx_v)

            @pl.when(c == 1)
            def _():
                pltpu.sync_copy(dst_hbm.at[g], idx_v)

            def body(j, carry2):
                pltpu.async_copy(ones_v, acc_sh.at[idx_v.at[j]], ssm, add=True)

                @pl.when(jnp.logical_or(j > 0, b > 0))
                def _():
                    pltpu.make_async_copy(
                        ones_v, acc_sh.at[idx_v.at[j]], ssm).wait()
                return carry2

            return lax.fori_loop(0, SB2, body, carry)

        lax.fori_loop(0, NB2F, blk, 0)
        pltpu.make_async_copy(ones_v, acc_sh.at[idx_v.at[0]], ssm).wait()
        plsc.subcore_barrier()
        pltpu.sync_copy(acc_sh.at[pl.ds(row0, RT)], out_hbm.at[c, pl.ds(row0, RT)])

    return deg


@functools.lru_cache(maxsize=None)
def _agg1_kernel():
    mesh = plsc.VectorSubcoreMesh(core_axis_name="c", subcore_axis_name="s")
    NB = 7

    @functools.partial(
        pl.kernel,
        out_type=jax.ShapeDtypeStruct((2, NP, 16), jnp.float32),
        mesh=mesh,
        compiler_params=pltpu.CompilerParams(use_tc_tiling_on_sc=False),
        scratch_types=[
            pltpu.VMEM((SB2, CH2), jnp.int32),
            pltpu.VMEM((SB2, CH2), jnp.int32),
            [pltpu.VMEM((CH2, 16), jnp.float32) for _ in range(NB)],
            pltpu.VMEM_SHARED((NP, 16), jnp.float32),
            [pltpu.SemaphoreType.DMA for _ in range(NB)],
            [pltpu.SemaphoreType.DMA for _ in range(NB)],
        ],
    )
    def agg1(xn_hbm, src_hbm, dst_hbm, zrow_hbm, out_hbm,
             src_v, dst_v, rbs, acc_sh, gsm, ssm):
        c = lax.axis_index("c")
        s = lax.axis_index("s")
        row0 = s * RT
        pltpu.sync_copy(zrow_hbm, acc_sh.at[pl.ds(row0, RT)])
        plsc.subcore_barrier()

        def blk(b, carry):
            g = (c * 16 + s) * NB2H + b
            pltpu.sync_copy(src_hbm.at[g], src_v)
            pltpu.sync_copy(dst_hbm.at[g], dst_v)
            for q in range(NB):
                pltpu.async_copy(xn_hbm.at[src_v.at[q]], rbs[q], gsm[q])

            def body(m, carry2):
                for q in range(NB):
                    j = NB * m + q
                    pltpu.make_async_copy(
                        xn_hbm.at[src_v.at[j]], rbs[q], gsm[q]).wait()
                    pltpu.async_copy(
                        rbs[q], acc_sh.at[dst_v.at[j]], ssm[q], add=True)
                    qp = (q - 1) % NB

                    @pl.when(jnp.logical_and(j >= 1, j + NB - 1 < SB2))
                    def _(j=j, q=q, qp=qp):
                        pltpu.make_async_copy(
                            rbs[qp], acc_sh.at[dst_v.at[j]], ssm[qp]).wait()
                        pltpu.async_copy(
                            xn_hbm.at[src_v.at[j + NB - 1]], rbs[qp], gsm[qp])
                return carry2

            lax.fori_loop(0, SB2 // NB, body, carry)
            for q in range(NB):
                pltpu.make_async_copy(
                    rbs[q], acc_sh.at[dst_v.at[q]], ssm[q]).wait()
            return carry

        lax.fori_loop(0, NB2H, blk, 0)
        plsc.subcore_barrier()
        pltpu.sync_copy(acc_sh.at[pl.ds(row0, RT)], out_hbm.at[c, pl.ds(row0, RT)])

    return agg1


@functools.lru_cache(maxsize=None)
def _agg2_kernel():
    mesh = plsc.VectorSubcoreMesh(core_axis_name="c", subcore_axis_name="s")
    NB = 7
    oshape = jax.ShapeDtypeStruct((8, NP, 16), jnp.float32)

    @functools.partial(
        pl.kernel,
        out_type=oshape,
        mesh=mesh,
        compiler_params=pltpu.CompilerParams(use_tc_tiling_on_sc=False),
        scratch_types=[
            pltpu.VMEM((SB2, CH2), jnp.int32),
            pltpu.VMEM((SB2, CH2), jnp.int32),
            [pltpu.VMEM((CH2, 16), jnp.float32) for _ in range(NB)],
            pltpu.VMEM_SHARED((NP, 16), jnp.float32),
            [pltpu.SemaphoreType.DMA for _ in range(NB)],
            [pltpu.SemaphoreType.DMA for _ in range(NB)],
        ],
    )
    def agg2(t8_hbm, src_hbm, dst_hbm, zrow_hbm, o8_hbm,
             src_v, dst_v, rbs, acc_sh, gsm, ssm):
        c = lax.axis_index("c")
        s = lax.axis_index("s")
        row0 = s * RT
        for cc in range(2):
            @pl.when(c == cc)
            def _(cc=cc):
                for kk in range(4):
                    k = 4 * cc + kk
                    tbl = t8_hbm.at[k]
                    pltpu.sync_copy(zrow_hbm, acc_sh.at[pl.ds(row0, RT)])
                    plsc.subcore_barrier()

                    def blk(b, carry, tbl=tbl):
                        g = s * NB2F + b
                        pltpu.sync_copy(src_hbm.at[g], src_v)
                        pltpu.sync_copy(dst_hbm.at[g], dst_v)
                        for q in range(NB):
                            pltpu.async_copy(tbl.at[src_v.at[q]], rbs[q], gsm[q])

                        def body(m, carry2, tbl=tbl):
                            for q in range(NB):
                                j = NB * m + q
                                pltpu.make_async_copy(
                                    tbl.at[src_v.at[j]], rbs[q], gsm[q]).wait()
                                pltpu.async_copy(
                                    rbs[q], acc_sh.at[dst_v.at[j]], ssm[q],
                                    add=True)
                                qp = (q - 1) % NB

                                @pl.when(jnp.logical_and(j >= 1,
                                                         j + NB - 1 < SB2))
                                def _(j=j, qp=qp, tbl=tbl):
                                    pltpu.make_async_copy(
                                        rbs[qp], acc_sh.at[dst_v.at[j]],
                                        ssm[qp]).wait()
                                    pltpu.async_copy(
                                        tbl.at[src_v.at[j + NB - 1]],
                                        rbs[qp], gsm[qp])
                            return carry2

                        lax.fori_loop(0, SB2 // NB, body, carry)
                        for q in range(NB):
                            pltpu.make_async_copy(
                                rbs[q], acc_sh.at[dst_v.at[q]], ssm[q]).wait()
                        return carry

                    lax.fori_loop(0, NB2F, blk, 0)
                    plsc.subcore_barrier()
                    pltpu.sync_copy(acc_sh.at[pl.ds(row0, RT)],
                                    o8_hbm.at[k, pl.ds(row0, RT)])
                    plsc.subcore_barrier()

    return agg2


def _prep_body(degs_ref, x_ref, xn_ref):
    ns = lax.rsqrt(jnp.maximum(degs_ref[0][:, 0:1], 1.0))
    xn_ref[...] = x_ref[...] * ns


def _prep_call(degs, xpad):
    return pl.pallas_call(
        _prep_body,
        grid=(16,),
        in_specs=[
            pl.BlockSpec((2, RT, 16), lambda i: (0, i, 0)),
            pl.BlockSpec((RT, 16), lambda i: (i, 0)),
        ],
        out_specs=pl.BlockSpec((RT, 16), lambda i: (i, 0)),
        out_shape=jax.ShapeDtypeStruct((NP, 16), jnp.float32),
    )(degs, xpad)


def _dense_body(aggp_ref, degs_ref, w1_ref, b1_ref, w2_ref, t_ref):
    ns = lax.rsqrt(jnp.maximum(degs_ref[0][:, 0:1], 1.0))
    nd = lax.rsqrt(jnp.maximum(degs_ref[1][:, 0:1], 1.0))
    agg = (aggp_ref[0] + aggp_ref[1]) * nd
    h1 = jnp.dot(agg, w1_ref[...], preferred_element_type=jnp.float32,
                 precision=lax.Precision.HIGHEST)
    h1 = jnp.maximum(h1 + b1_ref[...], 0.0)
    h1n = h1 * ns
    t_ref[...] = jnp.dot(h1n, w2_ref[...], preferred_element_type=jnp.float32,
                         precision=lax.Precision.HIGHEST)


def _dense_call(aggp, degs, w1p, b1r, W2):
    return pl.pallas_call(
        _dense_body,
        grid=(16,),
        in_specs=[
            pl.BlockSpec((2, RT, 16), lambda i: (0, i, 0)),
            pl.BlockSpec((2, RT, 16), lambda i: (0, i, 0)),
            pl.BlockSpec((16, FH1), lambda i: (0, 0)),
            pl.BlockSpec((1, FH1), lambda i: (0, 0)),
            pl.BlockSpec((FH1, FH2), lambda i: (0, 0)),
        ],
        out_specs=pl.BlockSpec((RT, FH2), lambda i: (i, 0)),
        out_shape=jax.ShapeDtypeStruct((NP, FH2), jnp.float32),
    )(aggp, degs, w1p, b1r, W2)


def _final_body(a_ref, degs_ref, b2_ref, wc_ref, bc_ref, out_ref, acc):
    i = pl.program_id(0)
    nd = lax.rsqrt(jnp.maximum(degs_ref[1][:, 0:1], 1.0))
    rows = RT * i + lax.broadcasted_iota(jnp.int32, (RT, 1), 0)
    mask = rows < NN
    for k in range(8):
        h2 = jnp.maximum(a_ref[k] * nd + b2_ref[k:k + 1, :], 0.0)
        h2 = jnp.where(mask, h2, 0.0)
        part = jnp.sum(h2, axis=0, keepdims=True)

        @pl.when(i == 0)
        def _(k=k, part=part):
            acc[k:k + 1, :] = part

        @pl.when(i > 0)
        def _(k=k, part=part):
            acc[k:k + 1, :] = acc[k:k + 1, :] + part

    @pl.when(i == 15)
    def _():
        lg = bc_ref[...]
        for k in range(8):
            hg = acc[k:k + 1, :] * (1.0 / NN)
            lg = lg + jnp.dot(hg, wc_ref[k], preferred_element_type=jnp.float32,
                              precision=lax.Precision.HIGHEST)
        out_ref[...] = lg


def _final_call(aggs, degs, b2r, Wc, bcr):
    return pl.pallas_call(
        _final_body,
        grid=(16,),
        in_specs=[
            pl.BlockSpec((8, RT, 16), lambda i: (0, i, 0)),
            pl.BlockSpec((2, RT, 16), lambda i: (0, i, 0)),
            pl.BlockSpec((8, 16), lambda i: (0, 0)),
            pl.BlockSpec((8, 16, NCLS), lambda i: (0, 0, 0)),
            pl.BlockSpec((1, NCLS), lambda i: (0, 0)),
        ],
        out_specs=pl.BlockSpec((1, NCLS), lambda i: (0, 0)),
        out_shape=jax.ShapeDtypeStruct((1, NCLS), jnp.float32),
        scratch_shapes=[pltpu.VMEM((8, 16), jnp.float32)],
    )(aggs, degs, b2r, Wc, bcr)


def kernel(x, edge_index, W1, b1, W2, b2, Wc, bc):
    src = edge_index[0]
    dst = edge_index[1]
    pad = jnp.full((EP - EE,), NN, dtype=jnp.int32)
    sp = jnp.concatenate([src, pad])
    dp = jnp.concatenate([dst, pad])
    src_w = sp.reshape(EP // (SB2 * CH2), SB2, CH2)
    dst_w = dp.reshape(EP // (SB2 * CH2), SB2, CH2)

    xpad = jnp.zeros((NP, 16), jnp.float32).at[:NN, :15].set(x)
    w1p = jnp.zeros((16, FH1), jnp.float32).at[:15].set(W1)
    z16 = jnp.zeros((RT, 16), jnp.float32)
    o16 = jnp.ones((CH2, 16), jnp.float32)

    degs = _deg_kernel()(src_w, dst_w, z16, o16)
    xn = _prep_call(degs, xpad)
    aggp = _agg1_kernel()(xn, src_w, dst_w, z16)
    t = _dense_call(aggp, degs, w1p, b1.reshape(1, FH1), W2)
    t8 = t.reshape(NP, 8, 16).transpose(1, 0, 2)
    aggs = _agg2_kernel()(t8, src_w, dst_w, z16)
    return _final_call(aggs, degs, b2.reshape(8, 16),
                       Wc.reshape(8, 16, NCLS), bc.reshape(1, NCLS))


# speedup vs baseline: 15.5051x; 1.1778x over previous
import functools

import jax
import jax.numpy as jnp
from jax import lax
from jax.experimental import pallas as pl
from jax.experimental.pallas import tpu as pltpu
from jax.experimental.pallas import tpu_sc as plsc

NN = 50000
NP = 50048
EE = 1600000
EP = 1605632
RT = 3128
CH = 128
NJF = 784
NJH = 392
FH1 = 256
FH2 = 128
NCLS = 10


CH2 = 256
SB2 = 14
NB2F = 28
NB2H = 14


@functools.lru_cache(maxsize=None)
def _deg_kernel():
    mesh = plsc.VectorSubcoreMesh(core_axis_name="c", subcore_axis_name="s")

    @functools.partial(
        pl.kernel,
        out_type=jax.ShapeDtypeStruct((2, NP, 16), jnp.float32),
        mesh=mesh,
        compiler_params=pltpu.CompilerParams(use_tc_tiling_on_sc=False),
        scratch_types=[
            pltpu.VMEM((SB2, CH2), jnp.int32),
            pltpu.VMEM((CH2, 16), jnp.float32),
            pltpu.VMEM_SHARED((NP, 16), jnp.float32),
            pltpu.SemaphoreType.DMA,
        ],
    )
    def deg(src_hbm, dst_hbm, zrow_hbm, ones_hbm, out_hbm, idx_v, ones_v,
            acc_sh, ssm):
        c = lax.axis_index("c")
        s = lax.axis_index("s")
        row0 = s * RT
        pltpu.sync_copy(zrow_hbm, acc_sh.at[pl.ds(row0, RT)])
        pltpu.sync_copy(ones_hbm, ones_v)
        plsc.subcore_barrier()

        def blk(b, carry):
            g = s * NB2F + b

            @pl.when(c == 0)
            def _():
                pltpu.sync_copy(src_hbm.at[g], idx_v)

            @pl.when(c == 1)
            def _():
                pltpu.sync_copy(dst_hbm.at[g], idx_v)

            def body(j, carry2):
                pltpu.async_copy(ones_v, acc_sh.at[idx_v.at[j]], ssm, add=True)

                @pl.when(jnp.logical_or(j > 0, b > 0))
                def _():
                    pltpu.make_async_copy(
                        ones_v, acc_sh.at[idx_v.at[j]], ssm).wait()
                return carry2

            return lax.fori_loop(0, SB2, body, carry)

        lax.fori_loop(0, NB2F, blk, 0)
        pltpu.make_async_copy(ones_v, acc_sh.at[idx_v.at[0]], ssm).wait()
        plsc.subcore_barrier()
        pltpu.sync_copy(acc_sh.at[pl.ds(row0, RT)], out_hbm.at[c, pl.ds(row0, RT)])

    return deg


@functools.lru_cache(maxsize=None)
def _agg1_kernel():
    mesh = plsc.VectorSubcoreMesh(core_axis_name="c", subcore_axis_name="s")
    NB = 7

    @functools.partial(
        pl.kernel,
        out_type=jax.ShapeDtypeStruct((2, NP, 16), jnp.float32),
        mesh=mesh,
        compiler_params=pltpu.CompilerParams(use_tc_tiling_on_sc=False),
        scratch_types=[
            pltpu.VMEM((SB2, CH2), jnp.int32),
            pltpu.VMEM((SB2, CH2), jnp.int32),
            [pltpu.VMEM((CH2, 16), jnp.float32) for _ in range(NB)],
            pltpu.VMEM_SHARED((NP, 16), jnp.float32),
            [pltpu.SemaphoreType.DMA for _ in range(NB)],
            [pltpu.SemaphoreType.DMA for _ in range(NB)],
        ],
    )
    def agg1(xn_hbm, src_hbm, dst_hbm, zrow_hbm, out_hbm,
             src_v, dst_v, rbs, acc_sh, gsm, ssm):
        c = lax.axis_index("c")
        s = lax.axis_index("s")
        row0 = s * RT
        pltpu.sync_copy(zrow_hbm, acc_sh.at[pl.ds(row0, RT)])
        plsc.subcore_barrier()

        def blk(b, carry):
            g = (c * 16 + s) * NB2H + b
            pltpu.sync_copy(src_hbm.at[g], src_v)
            pltpu.sync_copy(dst_hbm.at[g], dst_v)
            for q in range(NB):
                pltpu.async_copy(xn_hbm.at[src_v.at[q]], rbs[q], gsm[q])

            def body(m, carry2):
                for q in range(NB):
                    j = NB * m + q
                    pltpu.make_async_copy(
                        xn_hbm.at[src_v.at[j]], rbs[q], gsm[q]).wait()
                    pltpu.async_copy(
                        rbs[q], acc_sh.at[dst_v.at[j]], ssm[q], add=True)
                    qp = (q - 1) % NB

                    @pl.when(jnp.logical_and(j >= 1, j + NB - 1 < SB2))
                    def _(j=j, q=q, qp=qp):
                        pltpu.make_async_copy(
                            rbs[qp], acc_sh.at[dst_v.at[j]], ssm[qp]).wait()
                        pltpu.async_copy(
                            xn_hbm.at[src_v.at[j + NB - 1]], rbs[qp], gsm[qp])
                return carry2

            lax.fori_loop(0, SB2 // NB, body, carry)
            for q in range(NB):
                pltpu.make_async_copy(
                    rbs[q], acc_sh.at[dst_v.at[q]], ssm[q]).wait()
            return carry

        lax.fori_loop(0, NB2H, blk, 0)
        plsc.subcore_barrier()
        pltpu.sync_copy(acc_sh.at[pl.ds(row0, RT)], out_hbm.at[c, pl.ds(row0, RT)])

    return agg1


@functools.lru_cache(maxsize=None)
def _agg2_kernel():
    mesh = plsc.VectorSubcoreMesh(core_axis_name="c", subcore_axis_name="s")
    NB = 7
    oshape = jax.ShapeDtypeStruct((8, NP, 16), jnp.float32)

    @functools.partial(
        pl.kernel,
        out_type=oshape,
        mesh=mesh,
        compiler_params=pltpu.CompilerParams(use_tc_tiling_on_sc=False),
        scratch_types=[
            pltpu.VMEM((SB2, CH2), jnp.int32),
            pltpu.VMEM((SB2, CH2), jnp.int32),
            [pltpu.VMEM((CH2, 16), jnp.float32) for _ in range(NB)],
            pltpu.VMEM_SHARED((NP, 16), jnp.float32),
            [pltpu.SemaphoreType.DMA for _ in range(NB)],
            [pltpu.SemaphoreType.DMA for _ in range(NB)],
        ],
    )
    def agg2(tl_hbm, src_hbm, dst_hbm, zrow_hbm, o8_hbm,
             src_v, dst_v, rbs, acc_sh, gsm, ssm):
        c = lax.axis_index("c")
        s = lax.axis_index("s")
        row0 = s * RT
        for cc in range(2):
            @pl.when(c == cc)
            def _(cc=cc):
                for kk in range(4):
                    k = 4 * cc + kk
                    tbl = tl_hbm
                    pltpu.sync_copy(zrow_hbm, acc_sh.at[pl.ds(row0, RT)])
                    plsc.subcore_barrier()

                    def blk(b, carry, tbl=tbl, k=k):
                        g = s * NB2F + b
                        pltpu.sync_copy(src_hbm.at[g], src_v)
                        pltpu.sync_copy(dst_hbm.at[g], dst_v)
                        if k > 0:
                            def bias(r, cr):
                                for mm in range(CH2 // 16):
                                    sl = pl.ds(16 * mm, 16)
                                    src_v[r, sl] = src_v[r, sl] + k
                                return cr
                            lax.fori_loop(0, SB2, bias, 0)
                        for q in range(NB):
                            pltpu.async_copy(tbl.at[src_v.at[q]], rbs[q], gsm[q])

                        def body(m, carry2, tbl=tbl):
                            for q in range(NB):
                                j = NB * m + q
                                pltpu.make_async_copy(
                                    tbl.at[src_v.at[j]], rbs[q], gsm[q]).wait()
                                pltpu.async_copy(
                                    rbs[q], acc_sh.at[dst_v.at[j]], ssm[q],
                                    add=True)
                                qp = (q - 1) % NB

                                @pl.when(jnp.logical_and(j >= 1,
                                                         j + NB - 1 < SB2))
                                def _(j=j, qp=qp, tbl=tbl):
                                    pltpu.make_async_copy(
                                        rbs[qp], acc_sh.at[dst_v.at[j]],
                                        ssm[qp]).wait()
                                    pltpu.async_copy(
                                        tbl.at[src_v.at[j + NB - 1]],
                                        rbs[qp], gsm[qp])
                            return carry2

                        lax.fori_loop(0, SB2 // NB, body, carry)
                        for q in range(NB):
                            pltpu.make_async_copy(
                                rbs[q], acc_sh.at[dst_v.at[q]], ssm[q]).wait()
                        return carry

                    lax.fori_loop(0, NB2F, blk, 0)
                    plsc.subcore_barrier()
                    pltpu.sync_copy(acc_sh.at[pl.ds(row0, RT)],
                                    o8_hbm.at[k, pl.ds(row0, RT)])
                    plsc.subcore_barrier()

    return agg2


def _prep_body(degs_ref, degsp_ref, x_ref, xn_ref, ndil_ref):
    ns = lax.rsqrt(jnp.maximum(degs_ref[0][:, 0:1], 1.0))
    xn_ref[...] = x_ref[...] * ns

    @pl.when(pl.program_id(0) == 0)
    def _():
        ndil_ref[...] = lax.rsqrt(jnp.maximum(degsp_ref[1], 1.0))


def _prep_call(degs, degsp, xpad):
    return pl.pallas_call(
        _prep_body,
        grid=(16,),
        in_specs=[
            pl.BlockSpec((2, RT, 16), lambda i: (0, i, 0)),
            pl.BlockSpec((2, NP // 8, 128), lambda i: (0, 0, 0)),
            pl.BlockSpec((RT, 16), lambda i: (i, 0)),
        ],
        out_specs=[
            pl.BlockSpec((RT, 16), lambda i: (i, 0)),
            pl.BlockSpec((NP // 8, 128), lambda i: (0, 0)),
        ],
        out_shape=[
            jax.ShapeDtypeStruct((NP, 16), jnp.float32),
            jax.ShapeDtypeStruct((NP // 8, 128), jnp.float32),
        ],
    )(degs, degsp, xpad)


def _dense_body(aggp_ref, degs_ref, w1_ref, b1_ref, w2_ref, t_ref):
    ns = lax.rsqrt(jnp.maximum(degs_ref[0][:, 0:1], 1.0))
    nd = lax.rsqrt(jnp.maximum(degs_ref[1][:, 0:1], 1.0))
    agg = (aggp_ref[0] + aggp_ref[1]) * nd
    h1 = jnp.dot(agg, w1_ref[...], preferred_element_type=jnp.float32,
                 precision=lax.Precision.HIGHEST)
    h1 = jnp.maximum(h1 + b1_ref[...], 0.0)
    h1n = h1 * ns
    t_ref[...] = jnp.dot(h1n, w2_ref[...], preferred_element_type=jnp.float32,
                         precision=lax.Precision.HIGHEST)


def _dense_call(aggp, degs, w1p, b1r, W2):
    return pl.pallas_call(
        _dense_body,
        grid=(16,),
        in_specs=[
            pl.BlockSpec((2, RT, 16), lambda i: (0, i, 0)),
            pl.BlockSpec((2, RT, 16), lambda i: (0, i, 0)),
            pl.BlockSpec((16, FH1), lambda i: (0, 0)),
            pl.BlockSpec((1, FH1), lambda i: (0, 0)),
            pl.BlockSpec((FH1, FH2), lambda i: (0, 0)),
        ],
        out_specs=pl.BlockSpec((RT, FH2), lambda i: (i, 0)),
        out_shape=jax.ShapeDtypeStruct((NP, FH2), jnp.float32),
    )(aggp, degs, w1p, b1r, W2)


NBP = 368


def _final_body(a_ref, ndil_ref, b2il_ref, wf_ref, bc_ref, out_ref, acc):
    i = pl.program_id(0)
    rows = NBP * i + lax.broadcasted_iota(jnp.int32, (NBP, 1), 0)
    mask = rows < (NN // 8)
    nd = ndil_ref[...]
    for k in range(8):
        h2 = jnp.maximum(a_ref[k] * nd + b2il_ref[k], 0.0)
        h2 = jnp.where(mask, h2, 0.0)
        part = jnp.sum(h2, axis=0, keepdims=True)

        @pl.when(i == 0)
        def _(k=k, part=part):
            acc[k:k + 1, :] = part

        @pl.when(i > 0)
        def _(k=k, part=part):
            acc[k:k + 1, :] = acc[k:k + 1, :] + part

    @pl.when(i == 16)
    def _():
        lg = bc_ref[...]
        for k in range(8):
            lg = lg + jnp.dot(acc[k:k + 1, :] * (1.0 / NN), wf_ref[k],
                              preferred_element_type=jnp.float32,
                              precision=lax.Precision.HIGHEST)
        out_ref[...] = lg


def _final_call(a8, ndil, b2il, wfold, bcr):
    return pl.pallas_call(
        _final_body,
        grid=(17,),
        in_specs=[
            pl.BlockSpec((8, NBP, 128), lambda i: (0, i, 0)),
            pl.BlockSpec((NBP, 128), lambda i: (i, 0)),
            pl.BlockSpec((8, 1, 128), lambda i: (0, 0, 0)),
            pl.BlockSpec((8, 128, NCLS), lambda i: (0, 0, 0)),
            pl.BlockSpec((1, NCLS), lambda i: (0, 0)),
        ],
        out_specs=pl.BlockSpec((1, NCLS), lambda i: (0, 0)),
        out_shape=jax.ShapeDtypeStruct((1, NCLS), jnp.float32),
        scratch_shapes=[pltpu.VMEM((8, 128), jnp.float32)],
    )(a8, ndil, b2il, wfold, bcr)


def kernel(x, edge_index, W1, b1, W2, b2, Wc, bc):
    src = edge_index[0]
    dst = edge_index[1]
    pad = jnp.full((EP - EE,), NN, dtype=jnp.int32)
    sp = jnp.concatenate([src, pad])
    dp = jnp.concatenate([dst, pad])
    src_w = sp.reshape(EP // (SB2 * CH2), SB2, CH2)
    dst_w = dp.reshape(EP // (SB2 * CH2), SB2, CH2)
    src8_w = (sp * 8).reshape(EP // (SB2 * CH2), SB2, CH2)

    xpad = jnp.zeros((NP, 16), jnp.float32).at[:NN, :15].set(x)
    w1p = jnp.zeros((16, FH1), jnp.float32).at[:15].set(W1)
    z16 = jnp.zeros((RT, 16), jnp.float32)
    o16 = jnp.ones((CH2, 16), jnp.float32)

    degs = _deg_kernel()(src_w, dst_w, z16, o16)
    degsp = degs.reshape(2, NP // 8, 128)
    xn, ndil = _prep_call(degs, degsp, xpad)
    aggp = _agg1_kernel()(xn, src_w, dst_w, z16)
    t = _dense_call(aggp, degs, w1p, b1.reshape(1, FH1), W2)
    t_lin = t.reshape(8 * NP, 16)
    aggs = _agg2_kernel()(t_lin, src8_w, dst_w, z16)
    a8 = aggs.reshape(8, NP // 8, 128)
    b2il = jnp.broadcast_to(b2.reshape(8, 1, 16), (8, 8, 16)).reshape(8, 1, 128)
    wfold = jnp.broadcast_to(Wc.reshape(8, 1, 16, NCLS),
                             (8, 8, 16, NCLS)).reshape(8, 128, NCLS)
    return _final_call(a8, ndil, b2il, wfold, bc.reshape(1, NCLS))
